# baseline (speedup 1.0000x reference)
import jax
import jax.numpy as jnp
from jax.experimental import pallas as pl

_TOP_K = 2
_E = 8
_H = 768
_BLK = 2048


def _router_block(x_ref, w_ref, logits_ref, vals_ref, idx_ref):
    x = x_ref[...]
    w = w_ref[...]
    logits = jax.lax.dot_general(
        x, w, (((1,), (1,)), ((), ())), preferred_element_type=jnp.float32
    )
    logits_ref[...] = logits

    blk = logits.shape[0]
    lane = jax.lax.broadcasted_iota(jnp.int32, (blk, _E), 1)
    m1 = jnp.max(logits, axis=1, keepdims=True)
    i1 = jnp.min(jnp.where(logits == m1, lane, _E), axis=1, keepdims=True)
    masked = jnp.where(lane == i1, -jnp.inf, logits)
    m2 = jnp.max(masked, axis=1, keepdims=True)
    i2 = jnp.min(jnp.where(masked == m2, lane, _E), axis=1, keepdims=True)

    p1 = 1.0 / (1.0 + jnp.exp(m2 - m1))
    vals_ref[...] = jnp.concatenate([p1, 1.0 - p1], axis=1)
    idx_ref[...] = jnp.concatenate([i1, i2], axis=1)


def kernel(hidden_states, weight):
    hs = hidden_states.reshape(-1, _H)
    tokens = hs.shape[0]
    grid = (tokens // _BLK,)
    logits, vals, idx = pl.pallas_call(
        _router_block,
        grid=grid,
        in_specs=[
            pl.BlockSpec((_BLK, _H), lambda i: (i, 0)),
            pl.BlockSpec((_E, _H), lambda i: (0, 0)),
        ],
        out_specs=[
            pl.BlockSpec((_BLK, _E), lambda i: (i, 0)),
            pl.BlockSpec((_BLK, _TOP_K), lambda i: (i, 0)),
            pl.BlockSpec((_BLK, _TOP_K), lambda i: (i, 0)),
        ],
        out_shape=[
            jax.ShapeDtypeStruct((tokens, _E), jnp.float32),
            jax.ShapeDtypeStruct((tokens, _TOP_K), jnp.float32),
            jax.ShapeDtypeStruct((tokens, _TOP_K), jnp.int32),
        ],
    )(hs, weight)
    return (logits, vals, idx)


# speedup vs baseline: 2.4408x; 2.4408x over previous
import jax
import jax.numpy as jnp
from jax.experimental import pallas as pl

_TOP_K = 2
_E = 8
_H = 768
_BLK = 2048


def _router_block(x_ref, w_ref, logits_ref, vals_ref, idx_ref):
    x = x_ref[...]
    w = w_ref[...]
    lt = jax.lax.dot_general(
        w, x, (((1,), (1,)), ((), ())), preferred_element_type=jnp.float32
    )
    logits_ref[...] = lt

    blk = lt.shape[1]
    sub = jax.lax.broadcasted_iota(jnp.int32, (_E, blk), 0)
    m1 = jnp.max(lt, axis=0, keepdims=True)
    i1 = jnp.min(jnp.where(lt == m1, sub, _E), axis=0, keepdims=True)
    masked = jnp.where(sub == i1, -jnp.inf, lt)
    m2 = jnp.max(masked, axis=0, keepdims=True)
    i2 = jnp.min(jnp.where(masked == m2, sub, _E), axis=0, keepdims=True)

    p1 = 1.0 / (1.0 + jnp.exp(m2 - m1))
    vals_ref[...] = jnp.concatenate([p1, 1.0 - p1], axis=0)
    idx_ref[...] = jnp.concatenate([i1, i2], axis=0)


def kernel(hidden_states, weight):
    hs = hidden_states.reshape(-1, _H)
    tokens = hs.shape[0]
    grid = (tokens // _BLK,)
    logits_t, vals_t, idx_t = pl.pallas_call(
        _router_block,
        grid=grid,
        in_specs=[
            pl.BlockSpec((_BLK, _H), lambda i: (i, 0)),
            pl.BlockSpec((_E, _H), lambda i: (0, 0)),
        ],
        out_specs=[
            pl.BlockSpec((_E, _BLK), lambda i: (0, i)),
            pl.BlockSpec((_TOP_K, _BLK), lambda i: (0, i)),
            pl.BlockSpec((_TOP_K, _BLK), lambda i: (0, i)),
        ],
        out_shape=[
            jax.ShapeDtypeStruct((_E, tokens), jnp.float32),
            jax.ShapeDtypeStruct((_TOP_K, tokens), jnp.float32),
            jax.ShapeDtypeStruct((_TOP_K, tokens), jnp.int32),
        ],
    )(hs, weight)
    return (logits_t.T, vals_t.T, idx_t.T)


# speedup vs baseline: 2.4819x; 1.0168x over previous
import jax
import jax.numpy as jnp
from jax.experimental import pallas as pl

_TOP_K = 2
_E = 8
_H = 768
_BLK = 4096


def _router_block(x_ref, w_ref, logits_ref, vals_ref, idx_ref):
    x = x_ref[...]
    w = w_ref[...]
    lt = jax.lax.dot_general(
        w, x, (((1,), (1,)), ((), ())), preferred_element_type=jnp.float32
    )
    logits_ref[...] = lt

    blk = lt.shape[1]
    sub = jax.lax.broadcasted_iota(jnp.int32, (_E, blk), 0)
    m1 = jnp.max(lt, axis=0, keepdims=True)
    i1 = jnp.min(jnp.where(lt == m1, sub, _E), axis=0, keepdims=True)
    masked = jnp.where(sub == i1, -jnp.inf, lt)
    m2 = jnp.max(masked, axis=0, keepdims=True)
    i2 = jnp.min(jnp.where(masked == m2, sub, _E), axis=0, keepdims=True)

    p1 = 1.0 / (1.0 + jnp.exp(m2 - m1))
    vals_ref[...] = jnp.concatenate([p1, 1.0 - p1], axis=0)
    idx_ref[...] = jnp.concatenate([i1, i2], axis=0)


def kernel(hidden_states, weight):
    hs = hidden_states.reshape(-1, _H)
    tokens = hs.shape[0]
    grid = (tokens // _BLK,)
    logits_t, vals_t, idx_t = pl.pallas_call(
        _router_block,
        grid=grid,
        in_specs=[
            pl.BlockSpec((_BLK, _H), lambda i: (i, 0)),
            pl.BlockSpec((_E, _H), lambda i: (0, 0)),
        ],
        out_specs=[
            pl.BlockSpec((_E, _BLK), lambda i: (0, i)),
            pl.BlockSpec((_TOP_K, _BLK), lambda i: (0, i)),
            pl.BlockSpec((_TOP_K, _BLK), lambda i: (0, i)),
        ],
        out_shape=[
            jax.ShapeDtypeStruct((_E, tokens), jnp.float32),
            jax.ShapeDtypeStruct((_TOP_K, tokens), jnp.float32),
            jax.ShapeDtypeStruct((_TOP_K, tokens), jnp.int32),
        ],
    )(hs, weight)
    return (logits_t.T, vals_t.T, idx_t.T)


# speedup vs baseline: 2.4920x; 1.0041x over previous
import jax
import jax.numpy as jnp
from jax.experimental import pallas as pl
from jax.experimental.pallas import tpu as pltpu

_TOP_K = 2
_E = 8
_H = 768
_BLK = 4096


def _router_block(x_ref, w_ref, logits_ref, vals_ref, idx_ref):
    x = x_ref[...]
    w = w_ref[...]
    lt = jax.lax.dot_general(
        w, x, (((1,), (1,)), ((), ())), preferred_element_type=jnp.float32
    )
    logits_ref[...] = lt

    blk = lt.shape[1]
    sub = jax.lax.broadcasted_iota(jnp.int32, (_E, blk), 0)
    m1 = jnp.max(lt, axis=0, keepdims=True)
    i1 = jnp.min(jnp.where(lt == m1, sub, _E), axis=0, keepdims=True)
    masked = jnp.where(sub == i1, -jnp.inf, lt)
    m2 = jnp.max(masked, axis=0, keepdims=True)
    i2 = jnp.min(jnp.where(masked == m2, sub, _E), axis=0, keepdims=True)

    p1 = 1.0 / (1.0 + jnp.exp(m2 - m1))
    vals_ref[...] = jnp.concatenate([p1, 1.0 - p1], axis=0)
    idx_ref[...] = jnp.concatenate([i1, i2], axis=0)


def kernel(hidden_states, weight):
    hs = hidden_states.reshape(-1, _H)
    tokens = hs.shape[0]
    grid = (tokens // _BLK,)
    logits_t, vals_t, idx_t = pl.pallas_call(
        _router_block,
        grid=grid,
        in_specs=[
            pl.BlockSpec((_BLK, _H), lambda i: (i, 0)),
            pl.BlockSpec((_E, _H), lambda i: (0, 0)),
        ],
        out_specs=[
            pl.BlockSpec((_E, _BLK), lambda i: (0, i)),
            pl.BlockSpec((_TOP_K, _BLK), lambda i: (0, i)),
            pl.BlockSpec((_TOP_K, _BLK), lambda i: (0, i)),
        ],
        out_shape=[
            jax.ShapeDtypeStruct((_E, tokens), jnp.float32),
            jax.ShapeDtypeStruct((_TOP_K, tokens), jnp.float32),
            jax.ShapeDtypeStruct((_TOP_K, tokens), jnp.int32),
        ],
        compiler_params=pltpu.CompilerParams(
            dimension_semantics=("parallel",),
        ),
    )(hs, weight)
    return (logits_t.T, vals_t.T, idx_t.T)
